# baseline (speedup 1.0000x reference)
import functools

import jax
import jax.numpy as jnp
from jax import lax
from jax.experimental import pallas as pl
from jax.experimental.pallas import tpu as pltpu
from jax.experimental.pallas import tpu_sc as plsc

_NN_K = 5
_FILTER_SCALE = 2.0
_SIGMA = 0.75
_EPS = 1e-10
_B, _P, _D = 4, 4096, 3
_BR = 256
_NB = _P // _BR

_NTILES = 32
_PPT = _B * _P // _NTILES
_TPB = _P // _PPT
_LANES = 16


def _knn_tc_body(rows_ref, cols_ref, nrows_ref, d_ref, i_ref, nh_ref):
    rows = rows_ref[0]
    xr, yr, zr = rows[:, 0:1], rows[:, 1:2], rows[:, 2:3]
    cols = cols_ref[0]
    xc, yc, zc = cols[0:1, :], cols[1:2, :], cols[2:3, :]
    sq_r = xr * xr + yr * yr + zr * zr
    sq_c = xc * xc + yc * yc + zc * zc
    pq = jax.lax.dot_general(
        rows.astype(jnp.bfloat16), cols.astype(jnp.bfloat16),
        (((1,), (0,)), ((), ())),
        preferred_element_type=jnp.float32)
    d2 = jnp.maximum(sq_r + sq_c - 2.0 * pq, 0.0)

    half = _P // 2
    a = d2[:, :half]
    b = d2[:, half:]
    ia = lax.broadcasted_iota(jnp.int32, (_BR, half), 1).astype(jnp.float32)
    ib = ia + jnp.float32(half)
    amask = a <= b
    work = jnp.minimum(a, b)
    cur = jnp.where(amask, ia, ib)
    lval = jnp.maximum(a, b)
    lidx = jnp.where(amask, ib, ia)
    big = jnp.float32(2.0 * _P)
    inf = jnp.float32(jnp.inf)
    for k in range(_NN_K + 1):
        m = jnp.min(work, axis=1, keepdims=True)
        idxf = jnp.min(jnp.where(work <= m, cur, big), axis=1,
                       keepdims=True)
        if k >= 1:
            d_ref[0, :, k - 1:k] = m
            i_ref[0, :, k - 1:k] = idxf.astype(jnp.int32)
        if k < _NN_K:
            e = cur == idxf
            work = jnp.where(e, lval, work)
            cur = jnp.where(e, lidx, cur)
            lval = jnp.where(e, inf, lval)

    nr = nrows_ref[0]
    nrm = jnp.sqrt(jnp.sum(nr * nr, axis=1, keepdims=True))
    nh_ref[0] = nr * (1.0 / jnp.where(nrm < _EPS, _EPS, nrm))


def _knn_tc(points, points_t, normals):
    return pl.pallas_call(
        _knn_tc_body,
        grid=(_B, _NB),
        in_specs=[
            pl.BlockSpec((1, _BR, _D), lambda b, rb: (b, rb, 0)),
            pl.BlockSpec((1, _D, _P), lambda b, rb: (b, 0, 0)),
            pl.BlockSpec((1, _BR, _D), lambda b, rb: (b, rb, 0)),
        ],
        out_specs=[
            pl.BlockSpec((1, _BR, _NN_K), lambda b, rb: (b, rb, 0)),
            pl.BlockSpec((1, _BR, _NN_K), lambda b, rb: (b, rb, 0)),
            pl.BlockSpec((1, _BR, _D), lambda b, rb: (b, rb, 0)),
        ],
        out_shape=[
            jax.ShapeDtypeStruct((_B, _P, _NN_K), jnp.float32),
            jax.ShapeDtypeStruct((_B, _P, _NN_K), jnp.int32),
            jax.ShapeDtypeStruct((_B, _P, _D), jnp.float32),
        ],
    )(points, points_t, normals)


_N = _B * _P


def _weights_sc_body(d_hbm, i_hbm, nh_hbm, p_hbm, out_hbm,
                     ntab, ptab, dk_v, ik_v, own_v, ownp_v, out_v):
    wid = lax.axis_index("s") * 2 + lax.axis_index("c")
    base = wid * _PPT
    b_off = (wid // _TPB) * _P

    pltpu.sync_copy(nh_hbm.at[pl.ds(b_off * _D, _P * _D)], ntab)
    pltpu.sync_copy(p_hbm.at[pl.ds(b_off * _D, _P * _D)], ptab)
    pltpu.sync_copy(d_hbm.at[pl.ds(base * _NN_K, _PPT * _NN_K)], dk_v)
    pltpu.sync_copy(i_hbm.at[pl.ds(base * _NN_K, _PPT * _NN_K)], ik_v)
    pltpu.sync_copy(nh_hbm.at[pl.ds(base * _D, _PPT * _D)], own_v)
    pltpu.sync_copy(p_hbm.at[pl.ds(base * _D, _PPT * _D)], ownp_v)

    inv_sig_n = 1.0 / (_SIGMA * _SIGMA)
    lane = lax.broadcasted_iota(jnp.int32, (_LANES,), 0)

    def chunk(i, carry):
        pid = i * _LANES + lane
        p3 = pid * _D
        p5 = pid * _NN_K
        ox = plsc.load_gather(own_v, [p3])
        oy = plsc.load_gather(own_v, [p3 + 1])
        oz = plsc.load_gather(own_v, [p3 + 2])
        px = plsc.load_gather(ownp_v, [p3])
        py = plsc.load_gather(ownp_v, [p3 + 1])
        pz = plsc.load_gather(ownp_v, [p3 + 2])
        d1 = plsc.load_gather(dk_v, [p5])
        s = d1 * (2.0 * _FILTER_SCALE * _FILTER_SCALE)
        s = jnp.where(s < _EPS, jnp.float32(_EPS), s)
        inv_sp = 1.0 / jnp.where(d1 < _EPS, jnp.float32(_EPS), d1)
        acc = jnp.zeros((_LANES,), jnp.float32)
        dk = d1
        for k in range(_NN_K):
            if k > 0:
                dk = plsc.load_gather(dk_v, [p5 + k])
            i3 = plsc.load_gather(ik_v, [p5 + k]) * _D
            gx = plsc.load_gather(ntab, [i3])
            gy = plsc.load_gather(ntab, [i3 + 1])
            gz = plsc.load_gather(ntab, [i3 + 2])
            qx = plsc.load_gather(ptab, [i3])
            qy = plsc.load_gather(ptab, [i3 + 1])
            qz = plsc.load_gather(ptab, [i3 + 2])
            w = jnp.maximum(1.0 - dk / s, 0.0)
            w = w * w
            w = w * w
            dx, dy, dz = gx - ox, gy - oy, gz - oz
            dn2 = dx * dx + dy * dy + dz * dz
            wn = jnp.exp(-dn2 * inv_sig_n)
            ux, uy, uz = qx - px, qy - py, qz - pz
            dp2 = ux * ux + uy * uy + uz * uz
            ws = jnp.exp(-dp2 * inv_sp)
            acc = acc + w * wn * ws * dk
        out_v[pl.ds(i * _LANES, _LANES)] = acc
        return carry

    lax.fori_loop(0, _PPT // _LANES, chunk, 0)
    pltpu.sync_copy(out_v, out_hbm.at[pl.ds(base, _PPT)])


def _weights_sc(d5, i5, nh, pf):
    mesh = plsc.VectorSubcoreMesh(core_axis_name="c", subcore_axis_name="s")
    f = pl.kernel(
        _weights_sc_body,
        out_type=jax.ShapeDtypeStruct((_N,), jnp.float32),
        mesh=mesh,
        compiler_params=pltpu.CompilerParams(needs_layout_passes=False),
        scratch_types=[
            pltpu.VMEM((_P * _D,), jnp.float32),
            pltpu.VMEM((_P * _D,), jnp.float32),
            pltpu.VMEM((_PPT * _NN_K,), jnp.float32),
            pltpu.VMEM((_PPT * _NN_K,), jnp.int32),
            pltpu.VMEM((_PPT * _D,), jnp.float32),
            pltpu.VMEM((_PPT * _D,), jnp.float32),
            pltpu.VMEM((_PPT,), jnp.float32),
        ],
    )
    return f(d5, i5, nh, pf)


def kernel(points, normals):
    points_t = jnp.transpose(points, (0, 2, 1))
    d5, i5, nh = _knn_tc(points, points_t, normals)
    d5f = jnp.reshape(d5, (_NN_K * _N,))
    i5f = jnp.reshape(i5, (_NN_K * _N,))
    nhf = jnp.reshape(nh, (_D * _N,))
    pf = jnp.reshape(points, (_D * _N,))
    per_point = _weights_sc(d5f, i5f, nhf, pf)
    return jnp.mean(per_point)


# speedup vs baseline: 1.0062x; 1.0062x over previous
import jax
import jax.numpy as jnp
from jax import lax
from jax.experimental import pallas as pl
from jax.experimental.pallas import tpu as pltpu
from jax.experimental.pallas import tpu_sc as plsc

_NN_K = 5
_FILTER_SCALE = 2.0
_SIGMA = 0.75
_EPS = 1e-10
_B, _P, _D = 4, 4096, 3
_BR = 512
_NB = _P // _BR
_N = _B * _P

_NTILES = 32
_PPT = _N // _NTILES
_TPB = _P // _PPT
_LANES = 16


def _knn_tc_body(pts_ref, qts_ref, nts_ref, d_ref, i_ref, nh_ref):
    rb = pl.program_id(1)
    qsl = pl.ds(rb * _BR, _BR)
    cols = pts_ref[0]
    qs = qts_ref[0, :, qsl]
    sq_c = jnp.sum(cols * cols, axis=1, keepdims=True)
    xq, yq, zq = qs[0:1, :], qs[1:2, :], qs[2:3, :]
    sq_q = xq * xq + yq * yq + zq * zq
    pq = lax.dot_general(
        cols.astype(jnp.bfloat16), qs.astype(jnp.bfloat16),
        (((1,), (0,)), ((), ())),
        preferred_element_type=jnp.float32)
    d2 = jnp.maximum(sq_c + sq_q - 2.0 * pq, 0.0)

    half = _P // 2
    a = d2[:half, :]
    b = d2[half:, :]
    ia = lax.broadcasted_iota(jnp.int32, (half, _BR), 0).astype(jnp.float32)
    ib = ia + jnp.float32(half)
    amask = a <= b
    work = jnp.minimum(a, b)
    cur = jnp.where(amask, ia, ib)
    lval = jnp.maximum(a, b)
    lidx = jnp.where(amask, ib, ia)
    big = jnp.float32(2.0 * _P)
    inf = jnp.float32(jnp.inf)
    for k in range(_NN_K + 1):
        m = jnp.min(work, axis=0, keepdims=True)
        idxf = jnp.min(jnp.where(work <= m, cur, big), axis=0,
                       keepdims=True)
        if k >= 1:
            d_ref[0, k - 1:k, :] = m
            i_ref[0, k - 1:k, :] = idxf.astype(jnp.int32)
        if k < _NN_K:
            e = cur == idxf
            work = jnp.where(e, lval, work)
            cur = jnp.where(e, lidx, cur)
            lval = jnp.where(e, inf, lval)

    nr = nts_ref[0, :, qsl]
    nrm = jnp.sqrt(jnp.sum(nr * nr, axis=0, keepdims=True))
    nh_ref[0] = nr * (1.0 / jnp.where(nrm < _EPS, _EPS, nrm))


def _knn_tc(points, points_t, normals_t):
    return pl.pallas_call(
        _knn_tc_body,
        grid=(_B, _NB),
        in_specs=[
            pl.BlockSpec((1, _P, _D), lambda b, rb: (b, 0, 0)),
            pl.BlockSpec((1, _D, _P), lambda b, rb: (b, 0, 0)),
            pl.BlockSpec((1, _D, _P), lambda b, rb: (b, 0, 0)),
        ],
        out_specs=[
            pl.BlockSpec((1, _NN_K, _BR), lambda b, rb: (b, 0, rb)),
            pl.BlockSpec((1, _NN_K, _BR), lambda b, rb: (b, 0, rb)),
            pl.BlockSpec((1, _D, _BR), lambda b, rb: (b, 0, rb)),
        ],
        out_shape=[
            jax.ShapeDtypeStruct((_B, _NN_K, _P), jnp.float32),
            jax.ShapeDtypeStruct((_B, _NN_K, _P), jnp.int32),
            jax.ShapeDtypeStruct((_B, _D, _P), jnp.float32),
        ],
    )(points, points_t, normals_t)


def _weights_sc_body(d_hbm, i_hbm, nh_hbm, p_hbm, out_hbm, *scratch):
    ntabs = scratch[0:3]
    ptabs = scratch[3:6]
    owns = scratch[6:9]
    ownp = scratch[9:12]
    dks = scratch[12:17]
    iks = scratch[17:22]
    out_v = scratch[22]

    wid = lax.axis_index("s") * 2 + lax.axis_index("c")
    bb = wid // _TPB
    lo = (wid % _TPB) * _PPT

    for r in range(_D):
        pltpu.sync_copy(nh_hbm.at[pl.ds((bb * _D + r) * _P, _P)], ntabs[r])
        pltpu.sync_copy(p_hbm.at[pl.ds((bb * _D + r) * _P, _P)], ptabs[r])
        pltpu.sync_copy(nh_hbm.at[pl.ds((bb * _D + r) * _P + lo, _PPT)],
                        owns[r])
        pltpu.sync_copy(p_hbm.at[pl.ds((bb * _D + r) * _P + lo, _PPT)],
                        ownp[r])
    for k in range(_NN_K):
        pltpu.sync_copy(d_hbm.at[pl.ds((bb * _NN_K + k) * _P + lo, _PPT)],
                        dks[k])
        pltpu.sync_copy(i_hbm.at[pl.ds((bb * _NN_K + k) * _P + lo, _PPT)],
                        iks[k])

    inv_sig_n = 1.0 / (_SIGMA * _SIGMA)

    def chunk(i, carry):
        sl = pl.ds(i * _LANES, _LANES)
        ox, oy, oz = owns[0][sl], owns[1][sl], owns[2][sl]
        px, py, pz = ownp[0][sl], ownp[1][sl], ownp[2][sl]
        d1 = dks[0][sl]
        s = d1 * (2.0 * _FILTER_SCALE * _FILTER_SCALE)
        s = jnp.where(s < _EPS, jnp.float32(_EPS), s)
        inv_sp = 1.0 / jnp.where(d1 < _EPS, jnp.float32(_EPS), d1)
        acc = jnp.zeros((_LANES,), jnp.float32)
        for k in range(_NN_K):
            dk = dks[k][sl]
            idx = iks[k][sl]
            gx = plsc.load_gather(ntabs[0], [idx])
            gy = plsc.load_gather(ntabs[1], [idx])
            gz = plsc.load_gather(ntabs[2], [idx])
            qx = plsc.load_gather(ptabs[0], [idx])
            qy = plsc.load_gather(ptabs[1], [idx])
            qz = plsc.load_gather(ptabs[2], [idx])
            w = jnp.maximum(1.0 - dk / s, 0.0)
            w = w * w
            w = w * w
            dx, dy, dz = gx - ox, gy - oy, gz - oz
            dn2 = dx * dx + dy * dy + dz * dz
            wn = jnp.exp(-dn2 * inv_sig_n)
            ux, uy, uz = qx - px, qy - py, qz - pz
            dp2 = ux * ux + uy * uy + uz * uz
            ws = jnp.exp(-dp2 * inv_sp)
            acc = acc + w * wn * ws * dk
        out_v[sl] = acc
        return carry

    lax.fori_loop(0, _PPT // _LANES, chunk, 0)
    pltpu.sync_copy(out_v, out_hbm.at[pl.ds(wid * _PPT, _PPT)])


def _weights_sc(d5, i5, nh, pf):
    mesh = plsc.VectorSubcoreMesh(core_axis_name="c", subcore_axis_name="s")
    f = pl.kernel(
        _weights_sc_body,
        out_type=jax.ShapeDtypeStruct((_N,), jnp.float32),
        mesh=mesh,
        compiler_params=pltpu.CompilerParams(needs_layout_passes=False),
        scratch_types=(
            [pltpu.VMEM((_P,), jnp.float32)] * 6
            + [pltpu.VMEM((_PPT,), jnp.float32)] * 6
            + [pltpu.VMEM((_PPT,), jnp.float32)] * _NN_K
            + [pltpu.VMEM((_PPT,), jnp.int32)] * _NN_K
            + [pltpu.VMEM((_PPT,), jnp.float32)]
        ),
    )
    return f(d5, i5, nh, pf)


def kernel(points, normals):
    points_t = jnp.transpose(points, (0, 2, 1))
    normals_t = jnp.transpose(normals, (0, 2, 1))
    d5, i5, nh = _knn_tc(points, points_t, normals_t)
    per_point = _weights_sc(
        jnp.reshape(d5, (-1,)), jnp.reshape(i5, (-1,)),
        jnp.reshape(nh, (-1,)), jnp.reshape(points_t, (-1,)))
    return jnp.mean(per_point)


# speedup vs baseline: 1.0320x; 1.0257x over previous
import functools

import jax
import jax.numpy as jnp
from jax import lax
from jax.experimental import pallas as pl
from jax.experimental.pallas import tpu as pltpu
from jax.experimental.pallas import tpu_sc as plsc

_NN_K = 5
_FILTER_SCALE = 2.0
_SIGMA = 0.75
_EPS = 1e-10
_B, _P, _D = 4, 4096, 3
_BR = 512
_NB = _P // _BR

_NTILES = 32
_PPT = _B * _P // _NTILES
_TPB = _P // _PPT
_LANES = 16


def _knn_tc_body(rows_ref, cols_ref, nrows_ref, d_ref, i_ref, nh_ref):
    rows = rows_ref[0]
    xr, yr, zr = rows[:, 0:1], rows[:, 1:2], rows[:, 2:3]
    cols = cols_ref[0]
    xc, yc, zc = cols[0:1, :], cols[1:2, :], cols[2:3, :]
    sq_r = xr * xr + yr * yr + zr * zr
    sq_c = xc * xc + yc * yc + zc * zc
    pq = jax.lax.dot_general(
        rows.astype(jnp.bfloat16), cols.astype(jnp.bfloat16),
        (((1,), (0,)), ((), ())),
        preferred_element_type=jnp.float32)
    d2 = jnp.maximum(sq_r + sq_c - 2.0 * pq, 0.0)

    half = _P // 2
    a = d2[:, :half]
    b = d2[:, half:]
    ia = lax.broadcasted_iota(jnp.int32, (_BR, half), 1).astype(jnp.float32)
    ib = ia + jnp.float32(half)
    amask = a <= b
    work = jnp.minimum(a, b)
    cur = jnp.where(amask, ia, ib)
    lval = jnp.maximum(a, b)
    lidx = jnp.where(amask, ib, ia)
    big = jnp.float32(2.0 * _P)
    inf = jnp.float32(jnp.inf)
    for k in range(_NN_K + 1):
        m = jnp.min(work, axis=1, keepdims=True)
        idxf = jnp.min(jnp.where(work <= m, cur, big), axis=1,
                       keepdims=True)
        if k >= 1:
            d_ref[0, :, k - 1:k] = m
            i_ref[0, :, k - 1:k] = idxf.astype(jnp.int32)
        if k < _NN_K:
            e = cur == idxf
            work = jnp.where(e, lval, work)
            cur = jnp.where(e, lidx, cur)
            lval = jnp.where(e, inf, lval)

    nr = nrows_ref[0]
    nrm = jnp.sqrt(jnp.sum(nr * nr, axis=1, keepdims=True))
    nh_ref[0] = nr * (1.0 / jnp.where(nrm < _EPS, _EPS, nrm))


def _knn_tc(points, points_t, normals):
    return pl.pallas_call(
        _knn_tc_body,
        grid=(_B, _NB),
        in_specs=[
            pl.BlockSpec((1, _BR, _D), lambda b, rb: (b, rb, 0)),
            pl.BlockSpec((1, _D, _P), lambda b, rb: (b, 0, 0)),
            pl.BlockSpec((1, _BR, _D), lambda b, rb: (b, rb, 0)),
        ],
        out_specs=[
            pl.BlockSpec((1, _BR, _NN_K), lambda b, rb: (b, rb, 0)),
            pl.BlockSpec((1, _BR, _NN_K), lambda b, rb: (b, rb, 0)),
            pl.BlockSpec((1, _BR, _D), lambda b, rb: (b, rb, 0)),
        ],
        out_shape=[
            jax.ShapeDtypeStruct((_B, _P, _NN_K), jnp.float32),
            jax.ShapeDtypeStruct((_B, _P, _NN_K), jnp.int32),
            jax.ShapeDtypeStruct((_B, _P, _D), jnp.float32),
        ],
    )(points, points_t, normals)


_N = _B * _P


def _weights_sc_body(d_hbm, i_hbm, nh_hbm, p_hbm, out_hbm, *scratch):
    ntabs = scratch[0:3]
    ptabs = scratch[3:6]
    owns = scratch[6:9]
    ownp = scratch[9:12]
    dks = scratch[12:17]
    iks = scratch[17:22]
    out_v = scratch[22]

    wid = lax.axis_index("s") * 2 + lax.axis_index("c")
    base = wid * _PPT
    b_off = (wid // _TPB) * _P

    for r in range(_D):
        pltpu.sync_copy(nh_hbm.at[pl.ds(r * _N + b_off, _P)], ntabs[r])
        pltpu.sync_copy(nh_hbm.at[pl.ds(r * _N + base, _PPT)], owns[r])
        pltpu.sync_copy(p_hbm.at[pl.ds(r * _N + b_off, _P)], ptabs[r])
        pltpu.sync_copy(p_hbm.at[pl.ds(r * _N + base, _PPT)], ownp[r])
    for k in range(_NN_K):
        pltpu.sync_copy(d_hbm.at[pl.ds(k * _N + base, _PPT)], dks[k])
        pltpu.sync_copy(i_hbm.at[pl.ds(k * _N + base, _PPT)], iks[k])

    inv_sig_n = 1.0 / (_SIGMA * _SIGMA)

    def chunk(i, carry):
        sl = pl.ds(i * _LANES, _LANES)
        ox, oy, oz = owns[0][sl], owns[1][sl], owns[2][sl]
        px, py, pz = ownp[0][sl], ownp[1][sl], ownp[2][sl]
        d1 = dks[0][sl]
        s = d1 * (2.0 * _FILTER_SCALE * _FILTER_SCALE)
        s = jnp.where(s < _EPS, jnp.float32(_EPS), s)
        inv_sp = 1.0 / jnp.where(d1 < _EPS, jnp.float32(_EPS), d1)
        acc = jnp.zeros((_LANES,), jnp.float32)
        for k in range(_NN_K):
            dk = dks[k][sl]
            idx = iks[k][sl]
            gx = plsc.load_gather(ntabs[0], [idx])
            gy = plsc.load_gather(ntabs[1], [idx])
            gz = plsc.load_gather(ntabs[2], [idx])
            qx = plsc.load_gather(ptabs[0], [idx])
            qy = plsc.load_gather(ptabs[1], [idx])
            qz = plsc.load_gather(ptabs[2], [idx])
            w = jnp.maximum(1.0 - dk / s, 0.0)
            w = w * w
            w = w * w
            dx, dy, dz = gx - ox, gy - oy, gz - oz
            dn2 = dx * dx + dy * dy + dz * dz
            wn = jnp.exp(-dn2 * inv_sig_n)
            ux, uy, uz = qx - px, qy - py, qz - pz
            dp2 = ux * ux + uy * uy + uz * uz
            ws = jnp.exp(-dp2 * inv_sp)
            acc = acc + w * wn * ws * dk
        out_v[sl] = acc
        return carry

    lax.fori_loop(0, _PPT // _LANES, chunk, 0)
    pltpu.sync_copy(out_v, out_hbm.at[pl.ds(base, _PPT)])


def _weights_sc(d5, i5, nh, pf):
    mesh = plsc.VectorSubcoreMesh(core_axis_name="c", subcore_axis_name="s")
    f = pl.kernel(
        _weights_sc_body,
        out_type=jax.ShapeDtypeStruct((_N,), jnp.float32),
        mesh=mesh,
        compiler_params=pltpu.CompilerParams(needs_layout_passes=False),
        scratch_types=(
            [pltpu.VMEM((_P,), jnp.float32)] * 6
            + [pltpu.VMEM((_PPT,), jnp.float32)] * 6
            + [pltpu.VMEM((_PPT,), jnp.float32)] * _NN_K
            + [pltpu.VMEM((_PPT,), jnp.int32)] * _NN_K
            + [pltpu.VMEM((_PPT,), jnp.float32)]
        ),
    )
    return f(d5, i5, nh, pf)


def kernel(points, normals):
    points_t = jnp.transpose(points, (0, 2, 1))
    d5, i5, nh = _knn_tc(points, points_t, normals)
    d5f = jnp.reshape(jnp.transpose(d5, (2, 0, 1)), (_NN_K * _N,))
    i5f = jnp.reshape(jnp.transpose(i5, (2, 0, 1)), (_NN_K * _N,))
    nhf = jnp.reshape(jnp.transpose(nh, (2, 0, 1)), (_D * _N,))
    pf = jnp.reshape(jnp.transpose(points, (2, 0, 1)), (_D * _N,))
    per_point = _weights_sc(d5f, i5f, nhf, pf)
    return jnp.mean(per_point)
